# baseline (speedup 1.0000x reference)
import functools

import jax
import jax.numpy as jnp
import numpy as np
from jax import lax
from jax.experimental import pallas as pl
from jax.experimental.pallas import tpu as pltpu
from jax.experimental.pallas import tpu_sc as plsc

_NC = 2
_NS = 16
_NW = _NC * _NS
_LANES = 16

_SELU_ALPHA = 1.6732632423543772
_SELU_SCALE = 1.0507009873554805

_PERM = np.concatenate([np.arange(0, 64, 2), np.arange(1, 64, 2)])
_INV_PERM = np.argsort(_PERM)


def _sc_mesh():
    return plsc.VectorSubcoreMesh(core_axis_name="c", subcore_axis_name="s")


def _pack_bf16_pair(lo_f32, hi_f32):
    one = jnp.uint32(1)
    h7fff = jnp.uint32(0x7FFF)
    s16 = jnp.uint32(16)
    lo_b = lax.bitcast_convert_type(lo_f32, jnp.uint32)
    hi_b = lax.bitcast_convert_type(hi_f32, jnp.uint32)
    lo_r = (lo_b + h7fff + ((lo_b >> s16) & one)) >> s16
    hi_r = (hi_b + h7fff + ((hi_b >> s16) & one)) >> s16
    word = (hi_r << s16) | lo_r
    return lax.bitcast_convert_type(word, jnp.int32)


def _deg_partials(row, col, ew, n_pad):
    e = col.shape[0]
    epw = e // _NW

    @functools.partial(
        pl.kernel,
        out_type=(
            jax.ShapeDtypeStruct((_NW, n_pad), jnp.float32),
            jax.ShapeDtypeStruct((e,), jnp.int32),
        ),
        mesh=_sc_mesh(),
        compiler_params=pltpu.CompilerParams(needs_layout_passes=False),
        scratch_types=[
            pltpu.VMEM((epw,), jnp.int32),
            pltpu.VMEM((epw,), jnp.int32),
            pltpu.VMEM((epw,), jnp.float32),
            pltpu.VMEM((epw,), jnp.int32),
            pltpu.VMEM((n_pad,), jnp.float32),
            pltpu.SemaphoreType.DMA,
        ],
    )
    def deg_kernel(row_hbm, col_hbm, ew_hbm, parts_hbm, rc_hbm,
                   row_v, col_v, ew_v, rc_v, acc_v, sem):
        wid = lax.axis_index("s") * _NC + lax.axis_index("c")
        base = wid * epw
        pltpu.async_copy(row_hbm.at[pl.ds(base, epw)], row_v, sem)
        pltpu.async_copy(col_hbm.at[pl.ds(base, epw)], col_v, sem)
        pltpu.async_copy(ew_hbm.at[pl.ds(base, epw)], ew_v, sem)

        zeros = jnp.zeros((_LANES,), jnp.float32)

        def zbody(i, carry):
            acc_v[pl.ds(i * _LANES, _LANES)] = zeros
            return carry

        lax.fori_loop(0, n_pad // _LANES, zbody, 0)
        pltpu.make_async_copy(row_hbm.at[pl.ds(base, epw)], row_v, sem).wait()
        pltpu.make_async_copy(col_hbm.at[pl.ds(base, epw)], col_v, sem).wait()
        pltpu.make_async_copy(ew_hbm.at[pl.ds(base, epw)], ew_v, sem).wait()

        s16 = jnp.int32(16)

        def body(i, carry):
            sl = pl.ds(i * _LANES, _LANES)
            c = col_v[sl]
            w = ew_v[sl]
            plsc.addupdate_scatter(acc_v, [c], w)
            r = row_v[sl]
            rc_v[sl] = r | lax.shift_left(c, s16)
            return carry

        lax.fori_loop(0, epw // _LANES, body, 0)
        pltpu.sync_copy(acc_v, parts_hbm.at[wid])
        pltpu.sync_copy(rc_v, rc_hbm.at[pl.ds(base, epw)])

    return deg_kernel(row, col, ew)


def _edge_aggregate(gp, rc, ew, n_pad, feats):
    e = rc.shape[0]
    nhalf = 2
    ppw = 2
    ehalf_sz = e // nhalf
    eb = 16000
    nblk = ehalf_sz // eb
    npair = nblk // 2
    unroll = 2

    @functools.partial(
        pl.kernel,
        out_type=jax.ShapeDtypeStruct((nhalf, feats, n_pad), jnp.float32),
        mesh=_sc_mesh(),
        compiler_params=pltpu.CompilerParams(needs_layout_passes=False),
        scratch_types=(
            [pltpu.VMEM((n_pad,), jnp.int32) for _ in range(ppw)]
            + [pltpu.VMEM((n_pad,), jnp.float32) for _ in range(2 * ppw)]
            + [
                pltpu.VMEM((eb,), jnp.int32),
                pltpu.VMEM((eb,), jnp.float32),
                pltpu.VMEM((eb,), jnp.int32),
                pltpu.VMEM((eb,), jnp.float32),
                pltpu.SemaphoreType.DMA,
                pltpu.SemaphoreType.DMA,
                pltpu.SemaphoreType.DMA,
            ]
        ),
    )
    def msg_kernel(gp_hbm, rc_hbm, ew_hbm, out_hbm, *scratch):
        gp_vs = scratch[:ppw]
        acc_vs = scratch[ppw:3 * ppw]
        b0 = scratch[3 * ppw:3 * ppw + 2] + (scratch[3 * ppw + 4],)
        b1 = scratch[3 * ppw + 2:3 * ppw + 4] + (scratch[3 * ppw + 5],)
        gsem = scratch[3 * ppw + 6]
        bufs = (b0, b1)
        wid = lax.axis_index("s") * _NC + lax.axis_index("c")
        ehalf = wid % nhalf
        pbase = (wid // nhalf) * ppw
        ebase0 = ehalf * ehalf_sz
        for p in range(ppw):
            pltpu.async_copy(gp_hbm.at[pbase + p], gp_vs[p], gsem)

        zeros = jnp.zeros((_LANES,), jnp.float32)

        def zbody(i, carry):
            for a in range(2 * ppw):
                acc_vs[a][pl.ds(i * _LANES, _LANES)] = zeros
            return carry

        lax.fori_loop(0, n_pad // _LANES, zbody, 0)
        for p in range(ppw):
            pltpu.make_async_copy(gp_hbm.at[pbase + p], gp_vs[p], gsem).wait()

        def start(slot, b):
            rcv, wv, sem = bufs[slot]
            ebase = ebase0 + b * eb
            pltpu.async_copy(rc_hbm.at[pl.ds(ebase, eb)], rcv, sem)
            pltpu.async_copy(ew_hbm.at[pl.ds(ebase, eb)], wv, sem)

        def wait(slot):
            rcv, wv, sem = bufs[slot]
            pltpu.make_async_copy(rc_hbm.at[pl.ds(0, eb)], rcv, sem).wait()
            pltpu.make_async_copy(ew_hbm.at[pl.ds(0, eb)], wv, sem).wait()

        mask16 = jnp.int32(0xFFFF)
        maskhi = jnp.int32(-65536)
        s16 = jnp.int32(16)

        def process(slot):
            rcv, wv, _ = bufs[slot]

            @plsc.parallel_loop(0, eb // _LANES, 1, unroll=unroll)
            def _chunks(i):
                sl = pl.ds(i * _LANES, _LANES)
                v_rc = rcv[sl]
                w = wv[sl]
                r = v_rc & mask16
                c = lax.shift_right_logical(v_rc, s16)
                for p in range(ppw):
                    vp = plsc.load_gather(gp_vs[p], [r])
                    lo = plsc.bitcast(lax.shift_left(vp, s16), jnp.float32)
                    hi = plsc.bitcast(vp & maskhi, jnp.float32)
                    plsc.addupdate_scatter(acc_vs[2 * p], [c], lo * w)
                    plsc.addupdate_scatter(acc_vs[2 * p + 1], [c], hi * w)

        start(0, 0)

        def pair(i, carry):
            start(1, 2 * i + 1)
            wait(0)
            process(0)

            @pl.when(i < npair - 1)
            def _():
                start(0, 2 * i + 2)

            wait(1)
            process(1)
            return carry

        lax.fori_loop(0, npair, pair, 0)
        half = feats // 2
        for p in range(ppw):
            pltpu.async_copy(acc_vs[2 * p], out_hbm.at[ehalf, pbase + p], gsem)
            pltpu.async_copy(acc_vs[2 * p + 1],
                             out_hbm.at[ehalf, half + pbase + p], gsem)
        for p in range(ppw):
            pltpu.make_async_copy(acc_vs[2 * p],
                                  out_hbm.at[ehalf, pbase + p], gsem).wait()
            pltpu.make_async_copy(acc_vs[2 * p + 1],
                                  out_hbm.at[ehalf, half + pbase + p],
                                  gsem).wait()

    return msg_kernel(gp, rc, ew)


def _embed_and_g1(x_t, w0_t, b0c, w1_e, w1_o, parts, n_pad):
    bn = 1280
    grid = n_pad // bn
    emb = w0_t.shape[0]
    half = w1_e.shape[0]
    in_ch = x_t.shape[0]
    nw = parts.shape[0]

    def body(x_ref, w0_ref, b0_ref, w1e_ref, w1o_ref, p_ref,
             dis_ref, g_ref, gp_ref):
        deg = jnp.sum(p_ref[...], axis=0, keepdims=True) + 1.0
        dis = lax.rsqrt(deg)
        dis_ref[...] = dis
        h = jnp.dot(w0_ref[...], x_ref[...], preferred_element_type=jnp.float32)
        h = h + b0_ref[...]
        h = _SELU_SCALE * jnp.where(h > 0, h, _SELU_ALPHA * (jnp.exp(h) - 1.0))
        ge = jnp.dot(w1e_ref[...], h, preferred_element_type=jnp.float32) * dis
        go = jnp.dot(w1o_ref[...], h, preferred_element_type=jnp.float32) * dis
        g_ref[...] = jnp.concatenate([ge, go], axis=0)
        gp_ref[...] = _pack_bf16_pair(ge, go)

    return pl.pallas_call(
        body,
        grid=(grid,),
        in_specs=[
            pl.BlockSpec((in_ch, bn), lambda i: (0, i)),
            pl.BlockSpec((emb, in_ch), lambda i: (0, 0)),
            pl.BlockSpec((emb, 1), lambda i: (0, 0)),
            pl.BlockSpec((half, emb), lambda i: (0, 0)),
            pl.BlockSpec((half, emb), lambda i: (0, 0)),
            pl.BlockSpec((nw, bn), lambda i: (0, i)),
        ],
        out_specs=(
            pl.BlockSpec((1, bn), lambda i: (0, i)),
            pl.BlockSpec((2 * half, bn), lambda i: (0, i)),
            pl.BlockSpec((half, bn), lambda i: (0, i)),
        ),
        out_shape=(
            jax.ShapeDtypeStruct((1, n_pad), jnp.float32),
            jax.ShapeDtypeStruct((2 * half, n_pad), jnp.float32),
            jax.ShapeDtypeStruct((half, n_pad), jnp.int32),
        ),
    )(x_t, w0_t, b0c, w1_e, w1_o, parts)


def _conv1_epilogue(s1, g1p, dis, w2_e, w2_o, b1c):
    feats, n_pad = g1p.shape
    half = feats // 2

    def body(s_ref, g_ref, d_ref, w2e_ref, w2o_ref, b1_ref,
             g2_ref, gp2_ref):
        d = d_ref[...]
        s = s_ref[0] + s_ref[1]
        out1 = d * (s + g_ref[...]) + b1_ref[...]
        he = jnp.dot(w2e_ref[...], out1, preferred_element_type=jnp.float32) * d
        ho = jnp.dot(w2o_ref[...], out1, preferred_element_type=jnp.float32) * d
        g2_ref[...] = jnp.concatenate([he, ho], axis=0)
        gp2_ref[...] = _pack_bf16_pair(he, ho)

    return pl.pallas_call(
        body,
        out_shape=(
            jax.ShapeDtypeStruct((feats, n_pad), jnp.float32),
            jax.ShapeDtypeStruct((half, n_pad), jnp.int32),
        ),
    )(s1, g1p, dis, w2_e, w2_o, b1c)


def _conv2_softmax(s2, g2p, dis, b2c):
    feats, n_pad = g2p.shape

    def body(s_ref, g_ref, d_ref, b2_ref, out_ref):
        o = d_ref[...] * (s_ref[0] + s_ref[1] + g_ref[...]) + b2_ref[...]
        m = jnp.max(o, axis=0, keepdims=True)
        ex = jnp.exp(o - m)
        out_ref[...] = ex / jnp.sum(ex, axis=0, keepdims=True)

    return pl.pallas_call(
        body,
        out_shape=jax.ShapeDtypeStruct((feats, n_pad), jnp.float32),
    )(s2, g2p, dis, b2c)


def kernel(x, edge_index, edge_attr, W0, b0, W1, b1, W2, b2):
    n = x.shape[0]
    n_pad = 10240
    feats = W1.shape[1]

    row = edge_index[0]
    col = edge_index[1]
    ew = edge_attr

    x_t = jnp.pad(x.T, ((0, 0), (0, n_pad - n)))
    w0_t = W0.T
    w1_t = W1.T
    w2_tp = W2.T[:, _PERM]
    w1_e = w1_t[0::2]
    w1_o = w1_t[1::2]
    w2_e = w2_tp[0::2]
    w2_o = w2_tp[1::2]
    b0c = b0[:, None]
    b1c = b1[_PERM][:, None]
    b2c = b2[_PERM][:, None]

    parts, rc = _deg_partials(row, col, ew, n_pad)
    dis, g1p, gp1 = _embed_and_g1(x_t, w0_t, b0c, w1_e, w1_o, parts, n_pad)
    s1 = _edge_aggregate(gp1, rc, ew, n_pad, feats)
    g2p, gp2 = _conv1_epilogue(s1, g1p, dis, w2_e, w2_o, b1c)
    s2 = _edge_aggregate(gp2, rc, ew, n_pad, feats)
    out_p = _conv2_softmax(s2, g2p, dis, b2c)

    return out_p[_INV_PERM][:, :n].T


# speedup vs baseline: 1.0205x; 1.0205x over previous
import functools

import jax
import jax.numpy as jnp
import numpy as np
from jax import lax
from jax.experimental import pallas as pl
from jax.experimental.pallas import tpu as pltpu
from jax.experimental.pallas import tpu_sc as plsc

_NC = 2
_NS = 16
_NW = _NC * _NS
_LANES = 16

_SELU_ALPHA = 1.6732632423543772
_SELU_SCALE = 1.0507009873554805


def _sc_mesh():
    return plsc.VectorSubcoreMesh(core_axis_name="c", subcore_axis_name="s")


def _pack_bf16_pair(lo_f32, hi_f32):
    one = jnp.uint32(1)
    h7fff = jnp.uint32(0x7FFF)
    s16 = jnp.uint32(16)
    lo_b = lax.bitcast_convert_type(lo_f32, jnp.uint32)
    hi_b = lax.bitcast_convert_type(hi_f32, jnp.uint32)
    lo_r = (lo_b + h7fff + ((lo_b >> s16) & one)) >> s16
    hi_r = (hi_b + h7fff + ((hi_b >> s16) & one)) >> s16
    word = (hi_r << s16) | lo_r
    return lax.bitcast_convert_type(word, jnp.int32)


def _deg_partials(row, col, ew, n_pad):
    e = col.shape[0]
    epw = e // _NW

    @functools.partial(
        pl.kernel,
        out_type=(
            jax.ShapeDtypeStruct((_NW, n_pad), jnp.float32),
            jax.ShapeDtypeStruct((e,), jnp.int32),
        ),
        mesh=_sc_mesh(),
        compiler_params=pltpu.CompilerParams(needs_layout_passes=False),
        scratch_types=[
            pltpu.VMEM((epw,), jnp.int32),
            pltpu.VMEM((epw,), jnp.int32),
            pltpu.VMEM((epw,), jnp.float32),
            pltpu.VMEM((epw,), jnp.int32),
            pltpu.VMEM((n_pad,), jnp.float32),
            pltpu.SemaphoreType.DMA,
        ],
    )
    def deg_kernel(row_hbm, col_hbm, ew_hbm, parts_hbm, rc_hbm,
                   row_v, col_v, ew_v, rc_v, acc_v, sem):
        wid = lax.axis_index("s") * _NC + lax.axis_index("c")
        base = wid * epw
        pltpu.async_copy(row_hbm.at[pl.ds(base, epw)], row_v, sem)
        pltpu.async_copy(col_hbm.at[pl.ds(base, epw)], col_v, sem)
        pltpu.async_copy(ew_hbm.at[pl.ds(base, epw)], ew_v, sem)

        zeros = jnp.zeros((_LANES,), jnp.float32)

        def zbody(i, carry):
            acc_v[pl.ds(i * _LANES, _LANES)] = zeros
            return carry

        lax.fori_loop(0, n_pad // _LANES, zbody, 0)
        pltpu.make_async_copy(row_hbm.at[pl.ds(base, epw)], row_v, sem).wait()
        pltpu.make_async_copy(col_hbm.at[pl.ds(base, epw)], col_v, sem).wait()
        pltpu.make_async_copy(ew_hbm.at[pl.ds(base, epw)], ew_v, sem).wait()

        s16 = jnp.int32(16)

        def body(i, carry):
            sl = pl.ds(i * _LANES, _LANES)
            c = col_v[sl]
            w = ew_v[sl]
            plsc.addupdate_scatter(acc_v, [c], w)
            r = row_v[sl]
            rc_v[sl] = r | lax.shift_left(c, s16)
            return carry

        lax.fori_loop(0, epw // _LANES, body, 0)
        pltpu.sync_copy(acc_v, parts_hbm.at[wid])
        pltpu.sync_copy(rc_v, rc_hbm.at[pl.ds(base, epw)])

    return deg_kernel(row, col, ew)


def _edge_aggregate(gp, rc, ew, n_pad, feats):
    e = rc.shape[0]
    nhalf = 2
    ppw = 2
    ehalf_sz = e // nhalf
    eb = 8000
    nblk = ehalf_sz // eb
    npair = nblk // 2
    unroll = 2

    @functools.partial(
        pl.kernel,
        out_type=jax.ShapeDtypeStruct((nhalf, feats, n_pad), jnp.float32),
        mesh=_sc_mesh(),
        compiler_params=pltpu.CompilerParams(needs_layout_passes=False),
        scratch_types=(
            [pltpu.VMEM((n_pad,), jnp.int32) for _ in range(ppw)]
            + [pltpu.VMEM((n_pad,), jnp.float32) for _ in range(2 * ppw)]
            + [
                pltpu.VMEM((eb,), jnp.int32),
                pltpu.VMEM((eb,), jnp.float32),
                pltpu.VMEM((eb,), jnp.int32),
                pltpu.VMEM((eb,), jnp.float32),
                pltpu.SemaphoreType.DMA,
                pltpu.SemaphoreType.DMA,
                pltpu.SemaphoreType.DMA,
            ]
        ),
    )
    def msg_kernel(gp_hbm, rc_hbm, ew_hbm, out_hbm, *scratch):
        gp_vs = scratch[:ppw]
        acc_vs = scratch[ppw:3 * ppw]
        b0 = scratch[3 * ppw:3 * ppw + 2] + (scratch[3 * ppw + 4],)
        b1 = scratch[3 * ppw + 2:3 * ppw + 4] + (scratch[3 * ppw + 5],)
        gsem = scratch[3 * ppw + 6]
        bufs = (b0, b1)
        wid = lax.axis_index("s") * _NC + lax.axis_index("c")
        ehalf = wid % nhalf
        pbase = (wid // nhalf) * ppw
        ebase0 = ehalf * ehalf_sz
        for p in range(ppw):
            pltpu.async_copy(gp_hbm.at[pbase + p], gp_vs[p], gsem)

        zeros = jnp.zeros((_LANES,), jnp.float32)

        def zbody(i, carry):
            for a in range(2 * ppw):
                acc_vs[a][pl.ds(i * _LANES, _LANES)] = zeros
            return carry

        lax.fori_loop(0, n_pad // _LANES, zbody, 0)
        for p in range(ppw):
            pltpu.make_async_copy(gp_hbm.at[pbase + p], gp_vs[p], gsem).wait()

        def start(slot, b):
            rcv, wv, sem = bufs[slot]
            ebase = ebase0 + b * eb
            pltpu.async_copy(rc_hbm.at[pl.ds(ebase, eb)], rcv, sem)
            pltpu.async_copy(ew_hbm.at[pl.ds(ebase, eb)], wv, sem)

        def wait(slot):
            rcv, wv, sem = bufs[slot]
            pltpu.make_async_copy(rc_hbm.at[pl.ds(0, eb)], rcv, sem).wait()
            pltpu.make_async_copy(ew_hbm.at[pl.ds(0, eb)], wv, sem).wait()

        mask16 = jnp.int32(0xFFFF)
        maskhi = jnp.int32(-65536)
        s16 = jnp.int32(16)

        def process(slot):
            rcv, wv, _ = bufs[slot]

            @plsc.parallel_loop(0, eb // _LANES, 1, unroll=unroll)
            def _chunks(i):
                sl = pl.ds(i * _LANES, _LANES)
                v_rc = rcv[sl]
                w = wv[sl]
                r = v_rc & mask16
                c = lax.shift_right_logical(v_rc, s16)
                for p in range(ppw):
                    vp = plsc.load_gather(gp_vs[p], [r])
                    lo = plsc.bitcast(lax.shift_left(vp, s16), jnp.float32)
                    hi = plsc.bitcast(vp & maskhi, jnp.float32)
                    plsc.addupdate_scatter(acc_vs[2 * p], [c], lo * w)
                    plsc.addupdate_scatter(acc_vs[2 * p + 1], [c], hi * w)

        start(0, 0)

        def pair(i, carry):
            start(1, 2 * i + 1)
            wait(0)
            process(0)

            @pl.when(i < npair - 1)
            def _():
                start(0, 2 * i + 2)

            wait(1)
            process(1)
            return carry

        lax.fori_loop(0, npair, pair, 0)
        half = feats // 2
        for p in range(ppw):
            pltpu.async_copy(acc_vs[2 * p], out_hbm.at[ehalf, pbase + p], gsem)
            pltpu.async_copy(acc_vs[2 * p + 1],
                             out_hbm.at[ehalf, half + pbase + p], gsem)
        for p in range(ppw):
            pltpu.make_async_copy(acc_vs[2 * p],
                                  out_hbm.at[ehalf, pbase + p], gsem).wait()
            pltpu.make_async_copy(acc_vs[2 * p + 1],
                                  out_hbm.at[ehalf, half + pbase + p],
                                  gsem).wait()

    return msg_kernel(gp, rc, ew)


def _embed_and_g1(x_t, w0_t, b0c, w1_e, w1_o, parts, n_pad):
    bn = 1280
    grid = n_pad // bn
    emb = w0_t.shape[0]
    half = w1_e.shape[0]
    in_ch = x_t.shape[0]
    nw = parts.shape[0]

    def body(x_ref, w0_ref, b0_ref, w1e_ref, w1o_ref, p_ref,
             dis_ref, g_ref, gp_ref):
        deg = jnp.sum(p_ref[...], axis=0, keepdims=True) + 1.0
        dis = lax.rsqrt(deg)
        dis_ref[...] = dis
        h = jnp.dot(w0_ref[...], x_ref[...], preferred_element_type=jnp.float32)
        h = h + b0_ref[...]
        h = _SELU_SCALE * jnp.where(h > 0, h, _SELU_ALPHA * (jnp.exp(h) - 1.0))
        ge = jnp.dot(w1e_ref[...], h, preferred_element_type=jnp.float32) * dis
        go = jnp.dot(w1o_ref[...], h, preferred_element_type=jnp.float32) * dis
        g_ref[...] = jnp.concatenate([ge, go], axis=0)
        gp_ref[...] = _pack_bf16_pair(ge, go)

    return pl.pallas_call(
        body,
        grid=(grid,),
        in_specs=[
            pl.BlockSpec((in_ch, bn), lambda i: (0, i)),
            pl.BlockSpec((emb, in_ch), lambda i: (0, 0)),
            pl.BlockSpec((emb, 1), lambda i: (0, 0)),
            pl.BlockSpec((half, emb), lambda i: (0, 0)),
            pl.BlockSpec((half, emb), lambda i: (0, 0)),
            pl.BlockSpec((nw, bn), lambda i: (0, i)),
        ],
        out_specs=(
            pl.BlockSpec((1, bn), lambda i: (0, i)),
            pl.BlockSpec((2 * half, bn), lambda i: (0, i)),
            pl.BlockSpec((half, bn), lambda i: (0, i)),
        ),
        out_shape=(
            jax.ShapeDtypeStruct((1, n_pad), jnp.float32),
            jax.ShapeDtypeStruct((2 * half, n_pad), jnp.float32),
            jax.ShapeDtypeStruct((half, n_pad), jnp.int32),
        ),
    )(x_t, w0_t, b0c, w1_e, w1_o, parts)


def _conv1_epilogue(s1, g1p, dis, w2_e, w2_o, b1c):
    feats, n_pad = g1p.shape
    half = feats // 2

    def body(s_ref, g_ref, d_ref, w2e_ref, w2o_ref, b1_ref,
             g2_ref, gp2_ref):
        d = d_ref[...]
        s = s_ref[0] + s_ref[1]
        out1 = d * (s + g_ref[...]) + b1_ref[...]
        he = jnp.dot(w2e_ref[...], out1, preferred_element_type=jnp.float32) * d
        ho = jnp.dot(w2o_ref[...], out1, preferred_element_type=jnp.float32) * d
        g2_ref[...] = jnp.concatenate([he, ho], axis=0)
        gp2_ref[...] = _pack_bf16_pair(he, ho)

    return pl.pallas_call(
        body,
        out_shape=(
            jax.ShapeDtypeStruct((feats, n_pad), jnp.float32),
            jax.ShapeDtypeStruct((half, n_pad), jnp.int32),
        ),
    )(s1, g1p, dis, w2_e, w2_o, b1c)


def _conv2_softmax(s2, g2p, dis, b2c, n):
    feats, n_pad = g2p.shape
    bn = 2560
    grid = n_pad // bn

    def body(s_ref, g_ref, d_ref, b2_ref, out_ref):
        o = d_ref[...] * (s_ref[0] + s_ref[1] + g_ref[...]) + b2_ref[...]
        m = jnp.max(o, axis=0, keepdims=True)
        ex = jnp.exp(o - m)
        sm = ex / jnp.sum(ex, axis=0, keepdims=True)
        out_ref[...] = sm.T

    out = pl.pallas_call(
        body,
        grid=(grid,),
        in_specs=[
            pl.BlockSpec((2, feats, bn), lambda i: (0, 0, i)),
            pl.BlockSpec((feats, bn), lambda i: (0, i)),
            pl.BlockSpec((1, bn), lambda i: (0, i)),
            pl.BlockSpec((feats, 1), lambda i: (0, 0)),
        ],
        out_specs=pl.BlockSpec((bn, feats), lambda i: (i, 0)),
        out_shape=jax.ShapeDtypeStruct((n_pad, feats), jnp.float32),
    )(s2, g2p, dis, b2c)
    return out[:n]


def kernel(x, edge_index, edge_attr, W0, b0, W1, b1, W2, b2):
    n = x.shape[0]
    n_pad = 10240
    feats = W1.shape[1]

    row = edge_index[0]
    col = edge_index[1]
    ew = edge_attr

    x_t = jnp.pad(x.T, ((0, 0), (0, n_pad - n)))
    w0_t = W0.T
    w1_t = W1.T
    w2_t = W2.T
    half = feats // 2
    w1_e = w1_t[:half]
    w1_o = w1_t[half:]
    w2_e = w2_t[:half]
    w2_o = w2_t[half:]
    b0c = b0[:, None]
    b1c = b1[:, None]
    b2c = b2[:, None]

    parts, rc = _deg_partials(row, col, ew, n_pad)
    dis, g1p, gp1 = _embed_and_g1(x_t, w0_t, b0c, w1_e, w1_o, parts, n_pad)
    s1 = _edge_aggregate(gp1, rc, ew, n_pad, feats)
    g2p, gp2 = _conv1_epilogue(s1, g1p, dis, w2_e, w2_o, b1c)
    s2 = _edge_aggregate(gp2, rc, ew, n_pad, feats)
    out = _conv2_softmax(s2, g2p, dis, b2c, n)

    return out


# speedup vs baseline: 1.0222x; 1.0017x over previous
import functools

import jax
import jax.numpy as jnp
import numpy as np
from jax import lax
from jax.experimental import pallas as pl
from jax.experimental.pallas import tpu as pltpu
from jax.experimental.pallas import tpu_sc as plsc

_NC = 2
_NS = 16
_NW = _NC * _NS
_LANES = 16

_SELU_ALPHA = 1.6732632423543772
_SELU_SCALE = 1.0507009873554805


def _sc_mesh():
    return plsc.VectorSubcoreMesh(core_axis_name="c", subcore_axis_name="s")


def _pack_bf16_pair(lo_f32, hi_f32):
    one = jnp.uint32(1)
    h7fff = jnp.uint32(0x7FFF)
    s16 = jnp.uint32(16)
    lo_b = lax.bitcast_convert_type(lo_f32, jnp.uint32)
    hi_b = lax.bitcast_convert_type(hi_f32, jnp.uint32)
    lo_r = (lo_b + h7fff + ((lo_b >> s16) & one)) >> s16
    hi_r = (hi_b + h7fff + ((hi_b >> s16) & one)) >> s16
    word = (hi_r << s16) | lo_r
    return lax.bitcast_convert_type(word, jnp.int32)


def _deg_partials(row, col, ew, n_pad):
    e = col.shape[0]
    epw = e // _NW

    @functools.partial(
        pl.kernel,
        out_type=(
            jax.ShapeDtypeStruct((_NW, n_pad), jnp.float32),
            jax.ShapeDtypeStruct((e,), jnp.int32),
        ),
        mesh=_sc_mesh(),
        compiler_params=pltpu.CompilerParams(needs_layout_passes=False),
        scratch_types=[
            pltpu.VMEM((epw,), jnp.int32),
            pltpu.VMEM((epw,), jnp.int32),
            pltpu.VMEM((epw,), jnp.float32),
            pltpu.VMEM((epw,), jnp.int32),
            pltpu.VMEM((n_pad,), jnp.float32),
            pltpu.SemaphoreType.DMA,
        ],
    )
    def deg_kernel(row_hbm, col_hbm, ew_hbm, parts_hbm, rc_hbm,
                   row_v, col_v, ew_v, rc_v, acc_v, sem):
        wid = lax.axis_index("s") * _NC + lax.axis_index("c")
        base = wid * epw
        pltpu.async_copy(row_hbm.at[pl.ds(base, epw)], row_v, sem)
        pltpu.async_copy(col_hbm.at[pl.ds(base, epw)], col_v, sem)
        pltpu.async_copy(ew_hbm.at[pl.ds(base, epw)], ew_v, sem)

        zeros = jnp.zeros((_LANES,), jnp.float32)

        def zbody(i, carry):
            acc_v[pl.ds(i * _LANES, _LANES)] = zeros
            return carry

        lax.fori_loop(0, n_pad // _LANES, zbody, 0)
        pltpu.make_async_copy(row_hbm.at[pl.ds(base, epw)], row_v, sem).wait()
        pltpu.make_async_copy(col_hbm.at[pl.ds(base, epw)], col_v, sem).wait()
        pltpu.make_async_copy(ew_hbm.at[pl.ds(base, epw)], ew_v, sem).wait()

        s16 = jnp.int32(16)

        def body(i, carry):
            sl = pl.ds(i * _LANES, _LANES)
            c = col_v[sl]
            w = ew_v[sl]
            plsc.addupdate_scatter(acc_v, [c], w)
            r = row_v[sl]
            rc_v[sl] = r | lax.shift_left(c, s16)
            return carry

        lax.fori_loop(0, epw // _LANES, body, 0)
        pltpu.sync_copy(acc_v, parts_hbm.at[wid])
        pltpu.sync_copy(rc_v, rc_hbm.at[pl.ds(base, epw)])

    return deg_kernel(row, col, ew)


def _edge_aggregate(gp, rc, ew, n_pad, feats):
    e = rc.shape[0]
    nhalf = 2
    ppw = 2
    ehalf_sz = e // nhalf
    eb = 8000
    nblk = ehalf_sz // eb
    npair = nblk // 2
    unroll = 2

    @functools.partial(
        pl.kernel,
        out_type=jax.ShapeDtypeStruct((nhalf, feats, n_pad), jnp.float32),
        mesh=_sc_mesh(),
        compiler_params=pltpu.CompilerParams(needs_layout_passes=False),
        scratch_types=(
            [pltpu.VMEM((n_pad,), jnp.int32) for _ in range(ppw)]
            + [pltpu.VMEM((n_pad,), jnp.float32) for _ in range(2 * ppw)]
            + [
                pltpu.VMEM((eb,), jnp.int32),
                pltpu.VMEM((eb,), jnp.float32),
                pltpu.VMEM((eb,), jnp.int32),
                pltpu.VMEM((eb,), jnp.float32),
                pltpu.SemaphoreType.DMA,
                pltpu.SemaphoreType.DMA,
                pltpu.SemaphoreType.DMA,
            ]
        ),
    )
    def msg_kernel(gp_hbm, rc_hbm, ew_hbm, out_hbm, *scratch):
        gp_vs = scratch[:ppw]
        acc_vs = scratch[ppw:3 * ppw]
        b0 = scratch[3 * ppw:3 * ppw + 2] + (scratch[3 * ppw + 4],)
        b1 = scratch[3 * ppw + 2:3 * ppw + 4] + (scratch[3 * ppw + 5],)
        gsem = scratch[3 * ppw + 6]
        bufs = (b0, b1)
        wid = lax.axis_index("s") * _NC + lax.axis_index("c")
        ehalf = wid % nhalf
        pbase = (wid // nhalf) * ppw
        ebase0 = ehalf * ehalf_sz
        for p in range(ppw):
            pltpu.async_copy(gp_hbm.at[pbase + p], gp_vs[p], gsem)

        zeros = jnp.zeros((_LANES,), jnp.float32)

        def zbody(i, carry):
            for a in range(2 * ppw):
                acc_vs[a][pl.ds(i * _LANES, _LANES)] = zeros
            return carry

        lax.fori_loop(0, n_pad // _LANES, zbody, 0)
        for p in range(ppw):
            pltpu.make_async_copy(gp_hbm.at[pbase + p], gp_vs[p], gsem).wait()

        def start(slot, b):
            rcv, wv, sem = bufs[slot]
            ebase = ebase0 + b * eb
            pltpu.async_copy(rc_hbm.at[pl.ds(ebase, eb)], rcv, sem)
            pltpu.async_copy(ew_hbm.at[pl.ds(ebase, eb)], wv, sem)

        def wait(slot):
            rcv, wv, sem = bufs[slot]
            pltpu.make_async_copy(rc_hbm.at[pl.ds(0, eb)], rcv, sem).wait()
            pltpu.make_async_copy(ew_hbm.at[pl.ds(0, eb)], wv, sem).wait()

        mask16 = jnp.int32(0xFFFF)
        maskhi = jnp.int32(-65536)
        s16 = jnp.int32(16)

        def process(slot):
            rcv, wv, _ = bufs[slot]

            @plsc.parallel_loop(0, eb // _LANES, 1, unroll=unroll)
            def _chunks(i):
                sl = pl.ds(i * _LANES, _LANES)
                v_rc = rcv[sl]
                w = wv[sl]
                r = v_rc & mask16
                c = lax.shift_right_logical(v_rc, s16)
                for p in range(ppw):
                    vp = plsc.load_gather(gp_vs[p], [r])
                    lo = plsc.bitcast(lax.shift_left(vp, s16), jnp.float32)
                    hi = plsc.bitcast(vp & maskhi, jnp.float32)
                    plsc.addupdate_scatter(acc_vs[2 * p], [c], lo * w)
                    plsc.addupdate_scatter(acc_vs[2 * p + 1], [c], hi * w)

        start(0, 0)

        def pair(i, carry):
            start(1, 2 * i + 1)
            wait(0)
            process(0)

            @pl.when(i < npair - 1)
            def _():
                start(0, 2 * i + 2)

            wait(1)
            process(1)
            return carry

        lax.fori_loop(0, npair, pair, 0)
        half = feats // 2
        for p in range(ppw):
            pltpu.async_copy(acc_vs[2 * p], out_hbm.at[ehalf, pbase + p], gsem)
            pltpu.async_copy(acc_vs[2 * p + 1],
                             out_hbm.at[ehalf, half + pbase + p], gsem)
        for p in range(ppw):
            pltpu.make_async_copy(acc_vs[2 * p],
                                  out_hbm.at[ehalf, pbase + p], gsem).wait()
            pltpu.make_async_copy(acc_vs[2 * p + 1],
                                  out_hbm.at[ehalf, half + pbase + p],
                                  gsem).wait()

    return msg_kernel(gp, rc, ew)


def _embed_and_g1(x, w0_t, b0c, w1_e, w1_o, parts, n_pad):
    bn = 1280
    grid = n_pad // bn
    emb = w0_t.shape[0]
    half = w1_e.shape[0]
    in_ch = x.shape[1]
    nw = parts.shape[0]

    def body(x_ref, w0_ref, b0_ref, w1e_ref, w1o_ref, p_ref,
             dis_ref, g_ref, gp_ref):
        deg = jnp.sum(p_ref[...], axis=0, keepdims=True) + 1.0
        dis = lax.rsqrt(deg)
        dis_ref[...] = dis
        h = jnp.dot(w0_ref[...], x_ref[...].T,
                    preferred_element_type=jnp.float32)
        h = h + b0_ref[...]
        h = _SELU_SCALE * jnp.where(h > 0, h, _SELU_ALPHA * (jnp.exp(h) - 1.0))
        ge = jnp.dot(w1e_ref[...], h, preferred_element_type=jnp.float32) * dis
        go = jnp.dot(w1o_ref[...], h, preferred_element_type=jnp.float32) * dis
        g_ref[...] = jnp.concatenate([ge, go], axis=0)
        gp_ref[...] = _pack_bf16_pair(ge, go)

    return pl.pallas_call(
        body,
        grid=(grid,),
        in_specs=[
            pl.BlockSpec((bn, in_ch), lambda i: (i, 0)),
            pl.BlockSpec((emb, in_ch), lambda i: (0, 0)),
            pl.BlockSpec((emb, 1), lambda i: (0, 0)),
            pl.BlockSpec((half, emb), lambda i: (0, 0)),
            pl.BlockSpec((half, emb), lambda i: (0, 0)),
            pl.BlockSpec((nw, bn), lambda i: (0, i)),
        ],
        out_specs=(
            pl.BlockSpec((1, bn), lambda i: (0, i)),
            pl.BlockSpec((2 * half, bn), lambda i: (0, i)),
            pl.BlockSpec((half, bn), lambda i: (0, i)),
        ),
        out_shape=(
            jax.ShapeDtypeStruct((1, n_pad), jnp.float32),
            jax.ShapeDtypeStruct((2 * half, n_pad), jnp.float32),
            jax.ShapeDtypeStruct((half, n_pad), jnp.int32),
        ),
    )(x, w0_t, b0c, w1_e, w1_o, parts)


def _conv1_epilogue(s1, g1p, dis, w2_e, w2_o, b1c):
    feats, n_pad = g1p.shape
    half = feats // 2

    def body(s_ref, g_ref, d_ref, w2e_ref, w2o_ref, b1_ref,
             g2_ref, gp2_ref):
        d = d_ref[...]
        s = s_ref[0] + s_ref[1]
        out1 = d * (s + g_ref[...]) + b1_ref[...]
        he = jnp.dot(w2e_ref[...], out1, preferred_element_type=jnp.float32) * d
        ho = jnp.dot(w2o_ref[...], out1, preferred_element_type=jnp.float32) * d
        g2_ref[...] = jnp.concatenate([he, ho], axis=0)
        gp2_ref[...] = _pack_bf16_pair(he, ho)

    return pl.pallas_call(
        body,
        out_shape=(
            jax.ShapeDtypeStruct((feats, n_pad), jnp.float32),
            jax.ShapeDtypeStruct((half, n_pad), jnp.int32),
        ),
    )(s1, g1p, dis, w2_e, w2_o, b1c)


def _conv2_softmax(s2, g2p, dis, b2c, n):
    feats, n_pad = g2p.shape
    bn = 2560
    grid = n_pad // bn

    def body(s_ref, g_ref, d_ref, b2_ref, out_ref):
        o = d_ref[...] * (s_ref[0] + s_ref[1] + g_ref[...]) + b2_ref[...]
        m = jnp.max(o, axis=0, keepdims=True)
        ex = jnp.exp(o - m)
        sm = ex / jnp.sum(ex, axis=0, keepdims=True)
        out_ref[...] = sm.T

    out = pl.pallas_call(
        body,
        grid=(grid,),
        in_specs=[
            pl.BlockSpec((2, feats, bn), lambda i: (0, 0, i)),
            pl.BlockSpec((feats, bn), lambda i: (0, i)),
            pl.BlockSpec((1, bn), lambda i: (0, i)),
            pl.BlockSpec((feats, 1), lambda i: (0, 0)),
        ],
        out_specs=pl.BlockSpec((bn, feats), lambda i: (i, 0)),
        out_shape=jax.ShapeDtypeStruct((n_pad, feats), jnp.float32),
    )(s2, g2p, dis, b2c)
    return out[:n]


def kernel(x, edge_index, edge_attr, W0, b0, W1, b1, W2, b2):
    n = x.shape[0]
    n_pad = 10240
    feats = W1.shape[1]

    row = edge_index[0]
    col = edge_index[1]
    ew = edge_attr

    w0_t = W0.T
    w1_t = W1.T
    w2_t = W2.T
    half = feats // 2
    w1_e = w1_t[:half]
    w1_o = w1_t[half:]
    w2_e = w2_t[:half]
    w2_o = w2_t[half:]
    b0c = b0[:, None]
    b1c = b1[:, None]
    b2c = b2[:, None]

    parts, rc = _deg_partials(row, col, ew, n_pad)
    dis, g1p, gp1 = _embed_and_g1(x, w0_t, b0c, w1_e, w1_o, parts, n_pad)
    s1 = _edge_aggregate(gp1, rc, ew, n_pad, feats)
    g2p, gp2 = _conv1_epilogue(s1, g1p, dis, w2_e, w2_o, b1c)
    s2 = _edge_aggregate(gp2, rc, ew, n_pad, feats)
    out = _conv2_softmax(s2, g2p, dis, b2c, n)

    return out
